# Initial kernel scaffold; baseline (speedup 1.0000x reference)
import jax
import jax.numpy as jnp
from jax.experimental import pallas as pl


def kernel(x, edge_index, batch, ratio):
    raise NotImplementedError("write your pallas kernel here")


# speedup vs baseline: 6.7252x; 6.7252x over previous
import functools

import jax
import jax.numpy as jnp
from jax import lax
from jax.experimental import pallas as pl
from jax.experimental.pallas import tpu as pltpu
from jax.experimental.pallas import tpu_sc as plsc

_N = 10000
_E = 320000
_D = 128
_NC = 2
_NS = 16
_NW = _NC * _NS
_EPW = _E // _NW
_K = 80
_NCHUNK = _EPW // _K
_NP = 10240
_RPT = _NP // _NS
_LANES = 8


def _sc_kernel_body(x_hbm, src_hbm, dst_hbm, acc_hbm, cnt_hbm,
                    sidx, didx, xs, xd, msg, ones_v, zrow,
                    acc_sh, cnt_sh, sem1, sem2):
    c = lax.axis_index("c")
    s = lax.axis_index("s")

    zf = jnp.zeros((16,), jnp.float32)
    of = jnp.ones((16,), jnp.float32)

    def _zrow(i, carry):
        for j in range(_LANES):
            msg[i, pl.ds(j * 16, 16)] = zf
        return carry
    lax.fori_loop(0, _K, _zrow, 0)
    for r in range(_RPT // _K):
        pltpu.sync_copy(msg, acc_sh.at[pl.ds(s * _RPT + r * _K, _K)])

    def _zcnt(i, carry):
        ones_v[pl.ds(i * 16, 16)] = of
        return carry
    lax.fori_loop(0, _K // 16, _zcnt, 0)

    def _zrest(i, carry):
        zrow[pl.ds(i * 16, 16)] = zf
        return carry
    lax.fori_loop(0, _RPT // 16, _zrest, 0)
    pltpu.sync_copy(zrow, cnt_sh.at[pl.ds(s * _RPT, _RPT)])

    plsc.subcore_barrier()

    ebase = c * (_E // _NC) + s * _EPW

    def _chunk(t, carry):
        eb = ebase + t * _K
        pltpu.sync_copy(src_hbm.at[pl.ds(eb, _K)], sidx)
        pltpu.sync_copy(dst_hbm.at[pl.ds(eb, _K)], didx)
        cp1 = pltpu.async_copy(x_hbm.at[sidx], xs, sem1)
        cp2 = pltpu.async_copy(x_hbm.at[didx], xd, sem2)
        cp1.wait()
        cp2.wait()

        def _edge(e, ecarry):
            diffs = []
            for j in range(_LANES):
                sl = pl.ds(j * 16, 16)
                diffs.append(xs[e, sl] - xd[e, sl])
            sq = diffs[0] * diffs[0]
            for j in range(1, _LANES):
                sq = sq + diffs[j] * diffs[j]
            tot = jnp.sum(sq)
            inv = jnp.full((16,), -1.0, jnp.float32) / jnp.broadcast_to(tot, (16,))
            for j in range(_LANES):
                msg[e, pl.ds(j * 16, 16)] = diffs[j] * inv
            return ecarry
        lax.fori_loop(0, _K, _edge, 0)

        pltpu.sync_copy(msg, acc_sh.at[didx], add=True)
        pltpu.sync_copy(ones_v, cnt_sh.at[didx], add=True)
        return carry
    lax.fori_loop(0, _NCHUNK, _chunk, 0)

    plsc.subcore_barrier()
    pltpu.sync_copy(acc_sh.at[pl.ds(s * _RPT, _RPT)],
                    acc_hbm.at[c, pl.ds(s * _RPT, _RPT)])
    pltpu.sync_copy(cnt_sh.at[pl.ds(s * _RPT, _RPT)],
                    cnt_hbm.at[pl.ds(c * _NP + s * _RPT, _RPT)])


_sc_kernel = functools.partial(
    pl.kernel,
    out_type=(
        jax.ShapeDtypeStruct((_NC, _NP, _D), jnp.float32),
        jax.ShapeDtypeStruct((_NC * _NP,), jnp.float32),
    ),
    mesh=plsc.VectorSubcoreMesh(core_axis_name="c", subcore_axis_name="s"),
    compiler_params=pltpu.CompilerParams(needs_layout_passes=False),
    scratch_types=[
        pltpu.VMEM((_K,), jnp.int32),
        pltpu.VMEM((_K,), jnp.int32),
        pltpu.VMEM((_K, _D), jnp.float32),
        pltpu.VMEM((_K, _D), jnp.float32),
        pltpu.VMEM((_K, _D), jnp.float32),
        pltpu.VMEM((_K,), jnp.float32),
        pltpu.VMEM((_RPT,), jnp.float32),
        pltpu.VMEM_SHARED((_NP, _D), jnp.float32),
        pltpu.VMEM_SHARED((_NP,), jnp.float32),
        pltpu.SemaphoreType.DMA,
        pltpu.SemaphoreType.DMA,
    ],
)(_sc_kernel_body)


_RB = 1024


def _fin_body(acc_ref, cnt_ref, ratio_ref, out_ref):
    p = acc_ref[0] + acc_ref[1]
    cnt = jnp.maximum(cnt_ref[0] + cnt_ref[1], 1.0)
    out_ref[...] = ratio_ref[0, 0] * p / cnt


def kernel(x, edge_index, batch, ratio):
    src = edge_index[0]
    dst = edge_index[1]
    acc, cnt = _sc_kernel(x, src, dst)
    cnt3 = cnt.reshape(_NC, _NP, 1)
    out = pl.pallas_call(
        _fin_body,
        grid=(_NP // _RB,),
        in_specs=[
            pl.BlockSpec((_NC, _RB, _D), lambda i: (0, i, 0)),
            pl.BlockSpec((_NC, _RB, 1), lambda i: (0, i, 0)),
            pl.BlockSpec((1, 1), lambda i: (0, 0)),
        ],
        out_specs=pl.BlockSpec((_RB, _D), lambda i: (i, 0)),
        out_shape=jax.ShapeDtypeStruct((_NP, _D), jnp.float32),
    )(acc, cnt3, ratio.reshape(1, 1))
    return out[:_N]
